# baseline (speedup 1.0000x reference)
import numpy as np
import jax
import jax.numpy as jnp
from jax import lax
from jax.experimental import pallas as pl
from jax.experimental.pallas import tpu as pltpu
from jax.experimental.pallas import tpu_sc as plsc

BETA1 = 0.9
BETA2 = 0.999
EPS = 1e-08
LR = 0.001

_M = 100000
_D = 64
_B = 16384
_NW = 32
_RPT = _M // _NW
_NB = 25
_RB = _RPT // _NB
_RBP = 128
_CHUNK = 128
_LN_B1 = float(np.log(BETA1))
_LN_B2 = float(np.log(BETA2))

_STEP_GATHER = np.minimum(
    3125 * np.arange(_NW)[:, None, None]
    + 125 * np.arange(_NB)[None, :, None]
    + np.arange(_RBP)[None, None, :],
    _M - 1,
).reshape(_NW, _NB * _RBP).astype(np.int32)


def _sqrt16(x):
    x = jnp.maximum(x, 1e-30)
    i = lax.bitcast_convert_type(x, jnp.int32)
    y = lax.bitcast_convert_type(jnp.int32(0x5F3759DF) - (i >> 1), jnp.float32)
    for _ in range(2):
        y = y * (1.5 - 0.5 * x * y * y)
    return x * y


def _sc_body(idx_hbm, grad_hbm, emb_hbm, step_hbm, mem_hbm, pow_hbm,
             oemb_hbm, ostep_hbm, omem_hbm, opow_hbm,
             idx_v, pos_v, sub_v, tch_v, cidx_v, cnt_v, step_v,
             acc_v, emb_v, mem_v, pow_v, gbuf_v):
    cid = lax.axis_index("c")
    sid = lax.axis_index("s")
    wid = sid * 2 + cid
    lo = wid * _RPT

    iota = lax.iota(jnp.int32, 16)
    zeros16 = jnp.zeros((16,), jnp.float32)
    ones16 = jnp.ones((16,), jnp.float32)
    izeros16 = jnp.zeros((16,), jnp.int32)

    pltpu.sync_copy(idx_hbm, idx_v.at[pl.ds(0, _B)])
    pltpu.sync_copy(step_hbm.at[wid], step_v.at[pl.ds(0, _NB * _RBP)])

    @pl.loop(0, (_NB * _RBP) // 16, step=8)
    def _(k0):
        for u in range(8):
            cnt_v[pl.ds((k0 + u) * 16, 16)] = zeros16

    @pl.loop(0, sub_v.shape[0] // 16, step=8)
    def _(k0):
        for u in range(8):
            sub_v[pl.ds((k0 + u) * 16, 16)] = izeros16

    def scan_body(g4, base):
        for u in range(4):
            g = g4 * 4 + u
            v = idx_v[pl.ds(g * 16, 16)]
            vl = v - lo
            m = (vl >= 0) & (vl < _RPT)
            mi = m.astype(jnp.int32)
            blk = vl // _RB
            slot = blk * _RBP + (vl - blk * _RB)
            plsc.addupdate_scatter(cnt_v, [slot], ones16, mask=m)
            rank = plsc.cumsum(mi) - mi
            plsc.store_scatter(pos_v, [base + rank], iota + g * 16, mask=m)
            base = base + plsc.all_reduce_population_count(m)
        return base

    n_own_v = lax.fori_loop(0, _B // 64, scan_body, izeros16)
    n_own = n_own_v[0]
    nvec4 = (n_own + 63) // 64

    @pl.loop(0, _NB)
    def _blk(b):
        blk_lo = lo + b * _RB
        pltpu.sync_copy(emb_hbm.at[pl.ds(blk_lo, _RB)], emb_v)
        pltpu.sync_copy(mem_hbm.at[pl.ds(blk_lo, _RB)], mem_v)
        pltpu.sync_copy(pow_hbm.at[pl.ds(blk_lo, _RB)], pow_v)

        def touch_body(k, base):
            off = b * _RBP + k * 16
            vc = cnt_v[pl.ds(off, 16)]
            m = vc > 0.0
            sv = step_v[pl.ds(off, 16)]
            step_v[pl.ds(off, 16)] = jnp.where(m, sv + 1.0, sv)
            mi = m.astype(jnp.int32)
            rank = plsc.cumsum(mi) - mi
            plsc.store_scatter(tch_v, [base + rank], iota + k * 16, mask=m)
            return base + plsc.all_reduce_population_count(m)

        n_t_v = lax.fori_loop(0, _RBP // 16, touch_body, izeros16)
        n_t = n_t_v[0]
        n_tg = (n_t + 15) // 16

        def zacc_body(t, _t):
            valid = (t * 16 + iota) < n_t_v
            lrv = tch_v[pl.ds(t * 16, 16)]
            for cc in range(_D):
                ccv = jnp.full((16,), cc, jnp.int32)
                plsc.store_scatter(acc_v, [lrv, ccv], zeros16, mask=valid)
            return _t

        lax.fori_loop(0, n_tg, zacc_body, 0)

        def filt(h4, base):
            for u in range(4):
                h = h4 * 4 + u
                pv = pos_v[pl.ds(h * 16, 16)]
                valid = (h * 16 + iota) < n_own_v
                rows = plsc.load_gather(idx_v, [pv], mask=valid)
                m2 = valid & (rows >= blk_lo) & (rows < blk_lo + _RB)
                mi = m2.astype(jnp.int32)
                rank = plsc.cumsum(mi) - mi
                plsc.store_scatter(sub_v, [base + rank], pv, mask=m2)
                base = base + plsc.all_reduce_population_count(m2)
            return base

        n_sub_v = lax.fori_loop(0, nvec4, filt, izeros16)
        n_sub = n_sub_v[0]

        def chunk_body(c, _c):
            @pl.loop(0, _CHUNK // 16, step=8)
            def _(k0):
                for u in range(8):
                    cidx_v[pl.ds((k0 + u) * 16, 16)] = (
                        sub_v[pl.ds(c * _CHUNK + (k0 + u) * 16, 16)])

            pltpu.sync_copy(grad_hbm.at[cidx_v], gbuf_v)

            @pl.loop(0, _CHUNK // 16)
            def _(e):
                ent = c * _CHUNK + e * 16
                valid = (ent + iota) < n_sub_v
                pv = sub_v[pl.ds(ent, 16)]
                rows = plsc.load_gather(idx_v, [pv], mask=valid)
                lr = rows - blk_lo
                gid = iota + e * 16
                for cc in range(_D):
                    ccv = jnp.full((16,), cc, jnp.int32)
                    vals = plsc.load_gather(gbuf_v, [gid, ccv])
                    plsc.addupdate_scatter(acc_v, [lr, ccv], vals, mask=valid)

            return _c

        nch = (n_sub + _CHUNK - 1) // _CHUNK
        lax.fori_loop(0, nch, chunk_body, 0)

        def rows_body(t, _t):
            valid = (t * 16 + iota) < n_t_v
            lrv = tch_v[pl.ds(t * 16, 16)]
            slots = lrv + b * _RBP
            cntv = plsc.load_gather(cnt_v, [slots], mask=valid)
            snew = plsc.load_gather(step_v, [slots], mask=valid)
            inv_c = ones16 / cntv
            c1 = 1.0 - jnp.exp(snew * _LN_B1)
            c2 = 1.0 - jnp.exp(snew * _LN_B2)

            @pl.loop(0, _D, step=16)
            def _(cc0):
                for u in range(16):
                    ccv = jnp.full((16,), cc0 + u, jnp.int32)
                    g = plsc.load_gather(acc_v, [lrv, ccv], mask=valid) * inv_c
                    mv = plsc.load_gather(mem_v, [lrv, ccv], mask=valid)
                    pv = plsc.load_gather(pow_v, [lrv, ccv], mask=valid)
                    um = BETA1 * mv + (1.0 - BETA1) * g
                    up = BETA2 * pv + (1.0 - BETA2) * (g * g)
                    std = LR * (um / c1) / (_sqrt16(up / c2) + EPS)
                    plsc.addupdate_scatter(emb_v, [lrv, ccv], -std, mask=valid)
                    plsc.store_scatter(mem_v, [lrv, ccv], um, mask=valid)
                    plsc.store_scatter(pow_v, [lrv, ccv], up, mask=valid)

            return _t

        lax.fori_loop(0, n_tg, rows_body, 0)

        pltpu.sync_copy(emb_v, oemb_hbm.at[pl.ds(blk_lo, _RB)])
        pltpu.sync_copy(mem_v, omem_hbm.at[pl.ds(blk_lo, _RB)])
        pltpu.sync_copy(pow_v, opow_hbm.at[pl.ds(blk_lo, _RB)])

    pltpu.sync_copy(step_v.at[pl.ds(0, _NB * _RBP)], ostep_hbm.at[wid])


def kernel(idx, grad, emb, state_step, state_mem, state_power):
    step2d = state_step[_STEP_GATHER]
    mesh = plsc.VectorSubcoreMesh(core_axis_name="c", subcore_axis_name="s")
    out_type = [
        jax.ShapeDtypeStruct((_M, _D), jnp.float32),
        jax.ShapeDtypeStruct((_NW, _NB * _RBP), jnp.float32),
        jax.ShapeDtypeStruct((_M, _D), jnp.float32),
        jax.ShapeDtypeStruct((_M, _D), jnp.float32),
    ]
    scratch = [
        pltpu.VMEM((_B + 16,), jnp.int32),
        pltpu.VMEM((_B + 256,), jnp.int32),
        pltpu.VMEM((_B + _CHUNK + 16,), jnp.int32),
        pltpu.VMEM((_RBP + 16,), jnp.int32),
        pltpu.VMEM((_CHUNK,), jnp.int32),
        pltpu.VMEM((_NB * _RBP + 16,), jnp.float32),
        pltpu.VMEM((_NB * _RBP + 16,), jnp.float32),
        pltpu.VMEM((_RB, _D), jnp.float32),
        pltpu.VMEM((_RB, _D), jnp.float32),
        pltpu.VMEM((_RB, _D), jnp.float32),
        pltpu.VMEM((_RB, _D), jnp.float32),
        pltpu.VMEM((_CHUNK, _D), jnp.float32),
    ]
    f = pl.kernel(_sc_body, out_type=out_type, mesh=mesh,
                  scratch_types=scratch,
                  compiler_params=pltpu.CompilerParams(
                      use_tc_tiling_on_sc=False,
                      needs_layout_passes=False))
    oemb, ostep2d, omem, opow = f(idx, grad, emb, step2d,
                                  state_mem, state_power)
    new_step = ostep2d.reshape(_NW, _NB, _RBP)[:, :, :_RB].reshape(_M)
    return oemb, new_step, omem, opow


# speedup vs baseline: 2.2646x; 2.2646x over previous
import numpy as np
import jax
import jax.numpy as jnp
from jax import lax
from jax.experimental import pallas as pl
from jax.experimental.pallas import tpu as pltpu
from jax.experimental.pallas import tpu_sc as plsc

BETA1 = 0.9
BETA2 = 0.999
EPS = 1e-08
LR = 0.001

_M = 100000
_D = 64
_B = 16384
_NW = 32
_RPT = _M // _NW
_NB = 25
_RB = _RPT // _NB
_RBP = 128
_CHUNK = 128
_LN_B1 = float(np.log(BETA1))
_LN_B2 = float(np.log(BETA2))

_STEP_GATHER = np.minimum(
    3125 * np.arange(_NW)[:, None, None]
    + 125 * np.arange(_NB)[None, :, None]
    + np.arange(_RBP)[None, None, :],
    _M - 1,
).reshape(_NW, _NB * _RBP).astype(np.int32)


def _sqrt16(x):
    x = jnp.maximum(x, 1e-30)
    i = lax.bitcast_convert_type(x, jnp.int32)
    y = lax.bitcast_convert_type(jnp.int32(0x5F3759DF) - (i >> 1), jnp.float32)
    for _ in range(2):
        y = y * (1.5 - 0.5 * x * y * y)
    return x * y


def _sc_body(idx_hbm, grad_hbm, emb_hbm, step_hbm, mem_hbm, pow_hbm,
             oemb_hbm, ostep_hbm, omem_hbm, opow_hbm,
             idx_v, pos_v, sub_v, tch_v, cidx_v, cnt_v, step_v,
             acc_v, emb_v, mem_v, pow_v, gbuf_v):
    cid = lax.axis_index("c")
    sid = lax.axis_index("s")
    wid = sid * 2 + cid
    lo = wid * _RPT

    iota = lax.iota(jnp.int32, 16)
    zeros16 = jnp.zeros((16,), jnp.float32)
    ones16 = jnp.ones((16,), jnp.float32)
    izeros16 = jnp.zeros((16,), jnp.int32)

    pltpu.sync_copy(idx_hbm, idx_v.at[pl.ds(0, _B)])
    pltpu.sync_copy(step_hbm.at[wid], step_v.at[pl.ds(0, _NB * _RBP)])

    @pl.loop(0, (_NB * _RBP) // 16, step=8)
    def _(k0):
        for u in range(8):
            cnt_v[pl.ds((k0 + u) * 16, 16)] = zeros16

    @pl.loop(0, sub_v.shape[0] // 16, step=8)
    def _(k0):
        for u in range(8):
            sub_v[pl.ds((k0 + u) * 16, 16)] = izeros16

    def scan_body(g4, base):
        for u in range(4):
            g = g4 * 4 + u
            v = idx_v[pl.ds(g * 16, 16)]
            vl = v - lo
            m = (vl >= 0) & (vl < _RPT)
            mi = m.astype(jnp.int32)
            blk = vl // _RB
            slot = blk * _RBP + (vl - blk * _RB)
            plsc.addupdate_scatter(cnt_v, [slot], ones16, mask=m)
            rank = plsc.cumsum(mi) - mi
            plsc.store_scatter(pos_v, [base + rank], iota + g * 16, mask=m)
            base = base + plsc.all_reduce_population_count(m)
        return base

    n_own_v = lax.fori_loop(0, _B // 64, scan_body, izeros16)
    n_own = n_own_v[0]
    nvec4 = (n_own + 63) // 64

    @pl.loop(0, _NB)
    def _blk(b):
        blk_lo = lo + b * _RB
        pltpu.sync_copy(emb_hbm.at[pl.ds(blk_lo, _RB)], emb_v)
        pltpu.sync_copy(mem_hbm.at[pl.ds(blk_lo, _RB)], mem_v)
        pltpu.sync_copy(pow_hbm.at[pl.ds(blk_lo, _RB)], pow_v)

        def touch_body(k, base):
            off = b * _RBP + k * 16
            vc = cnt_v[pl.ds(off, 16)]
            m = vc > 0.0
            sv = step_v[pl.ds(off, 16)]
            step_v[pl.ds(off, 16)] = jnp.where(m, sv + 1.0, sv)
            mi = m.astype(jnp.int32)
            rank = plsc.cumsum(mi) - mi
            plsc.store_scatter(tch_v, [base + rank], iota + k * 16, mask=m)
            return base + plsc.all_reduce_population_count(m)

        n_t_v = lax.fori_loop(0, _RBP // 16, touch_body, izeros16)
        n_t = n_t_v[0]
        n_tg = (n_t + 15) // 16

        def zacc_body(t, _t):
            valid = (t * 16 + iota) < n_t_v
            lrv = tch_v[pl.ds(t * 16, 16)]
            for cc in range(_D):
                ccv = jnp.full((16,), cc, jnp.int32)
                plsc.store_scatter(acc_v, [lrv, ccv], zeros16, mask=valid)
            return _t

        lax.fori_loop(0, n_tg, zacc_body, 0)

        def filt(h4, base):
            for u in range(4):
                h = h4 * 4 + u
                pv = pos_v[pl.ds(h * 16, 16)]
                valid = (h * 16 + iota) < n_own_v
                rows = plsc.load_gather(idx_v, [pv], mask=valid)
                m2 = valid & (rows >= blk_lo) & (rows < blk_lo + _RB)
                mi = m2.astype(jnp.int32)
                rank = plsc.cumsum(mi) - mi
                plsc.store_scatter(sub_v, [base + rank], pv, mask=m2)
                base = base + plsc.all_reduce_population_count(m2)
            return base

        n_sub_v = lax.fori_loop(0, nvec4, filt, izeros16)
        n_sub = n_sub_v[0]

        def chunk_body(c, _c):
            nsc = jnp.minimum(n_sub - c * _CHUNK + 15, _CHUNK) // 16

            def sub_dma(k, _k):
                cidx_v[pl.ds(0, 16)] = sub_v[pl.ds(c * _CHUNK + k * 16, 16)]
                pltpu.sync_copy(grad_hbm.at[cidx_v.at[pl.ds(0, 16)]],
                                gbuf_v.at[pl.ds(k * 16, 16)])
                return _k

            lax.fori_loop(0, nsc, sub_dma, 0)

            @pl.loop(0, _CHUNK // 16)
            def _(e):
                ent = c * _CHUNK + e * 16
                valid = (ent + iota) < n_sub_v
                pv = sub_v[pl.ds(ent, 16)]
                rows = plsc.load_gather(idx_v, [pv], mask=valid)
                lr = rows - blk_lo
                gid = iota + e * 16
                for cc in range(_D):
                    ccv = jnp.full((16,), cc, jnp.int32)
                    vals = plsc.load_gather(gbuf_v, [gid, ccv])
                    plsc.addupdate_scatter(acc_v, [lr, ccv], vals, mask=valid)

            return _c

        nch = (n_sub + _CHUNK - 1) // _CHUNK
        lax.fori_loop(0, nch, chunk_body, 0)

        def rows_body(t, _t):
            valid = (t * 16 + iota) < n_t_v
            lrv = tch_v[pl.ds(t * 16, 16)]
            slots = lrv + b * _RBP
            cntv = plsc.load_gather(cnt_v, [slots], mask=valid)
            snew = plsc.load_gather(step_v, [slots], mask=valid)
            inv_c = ones16 / cntv
            c1 = 1.0 - jnp.exp(snew * _LN_B1)
            c2 = 1.0 - jnp.exp(snew * _LN_B2)

            @pl.loop(0, _D, step=16)
            def _(cc0):
                for u in range(16):
                    ccv = jnp.full((16,), cc0 + u, jnp.int32)
                    g = plsc.load_gather(acc_v, [lrv, ccv], mask=valid) * inv_c
                    mv = plsc.load_gather(mem_v, [lrv, ccv], mask=valid)
                    pv = plsc.load_gather(pow_v, [lrv, ccv], mask=valid)
                    um = BETA1 * mv + (1.0 - BETA1) * g
                    up = BETA2 * pv + (1.0 - BETA2) * (g * g)
                    std = LR * (um / c1) / (_sqrt16(up / c2) + EPS)
                    plsc.addupdate_scatter(emb_v, [lrv, ccv], -std, mask=valid)
                    plsc.store_scatter(mem_v, [lrv, ccv], um, mask=valid)
                    plsc.store_scatter(pow_v, [lrv, ccv], up, mask=valid)

            return _t

        lax.fori_loop(0, n_tg, rows_body, 0)

        pltpu.sync_copy(emb_v, oemb_hbm.at[pl.ds(blk_lo, _RB)])
        pltpu.sync_copy(mem_v, omem_hbm.at[pl.ds(blk_lo, _RB)])
        pltpu.sync_copy(pow_v, opow_hbm.at[pl.ds(blk_lo, _RB)])

    pltpu.sync_copy(step_v.at[pl.ds(0, _NB * _RBP)], ostep_hbm.at[wid])


def kernel(idx, grad, emb, state_step, state_mem, state_power):
    step2d = state_step[_STEP_GATHER]
    mesh = plsc.VectorSubcoreMesh(core_axis_name="c", subcore_axis_name="s")
    out_type = [
        jax.ShapeDtypeStruct((_M, _D), jnp.float32),
        jax.ShapeDtypeStruct((_NW, _NB * _RBP), jnp.float32),
        jax.ShapeDtypeStruct((_M, _D), jnp.float32),
        jax.ShapeDtypeStruct((_M, _D), jnp.float32),
    ]
    scratch = [
        pltpu.VMEM((_B + 16,), jnp.int32),
        pltpu.VMEM((_B + 256,), jnp.int32),
        pltpu.VMEM((_B + _CHUNK + 16,), jnp.int32),
        pltpu.VMEM((_RBP + 16,), jnp.int32),
        pltpu.VMEM((_CHUNK,), jnp.int32),
        pltpu.VMEM((_NB * _RBP + 16,), jnp.float32),
        pltpu.VMEM((_NB * _RBP + 16,), jnp.float32),
        pltpu.VMEM((_RB, _D), jnp.float32),
        pltpu.VMEM((_RB, _D), jnp.float32),
        pltpu.VMEM((_RB, _D), jnp.float32),
        pltpu.VMEM((_RB, _D), jnp.float32),
        pltpu.VMEM((_CHUNK, _D), jnp.float32),
    ]
    f = pl.kernel(_sc_body, out_type=out_type, mesh=mesh,
                  scratch_types=scratch,
                  compiler_params=pltpu.CompilerParams(
                      use_tc_tiling_on_sc=False,
                      needs_layout_passes=False))
    oemb, ostep2d, omem, opow = f(idx, grad, emb, step2d,
                                  state_mem, state_power)
    new_step = ostep2d.reshape(_NW, _NB, _RBP)[:, :, :_RB].reshape(_M)
    return oemb, new_step, omem, opow


# speedup vs baseline: 2.3493x; 1.0374x over previous
import numpy as np
import jax
import jax.numpy as jnp
from jax import lax
from jax.experimental import pallas as pl
from jax.experimental.pallas import tpu as pltpu
from jax.experimental.pallas import tpu_sc as plsc

BETA1 = 0.9
BETA2 = 0.999
EPS = 1e-08
LR = 0.001

_M = 100000
_D = 64
_B = 16384
_NW = 32
_RPT = _M // _NW
_NB = 25
_RB = _RPT // _NB
_RBP = 128
_CHUNK = 128
_LN_B1 = float(np.log(BETA1))
_LN_B2 = float(np.log(BETA2))

_STEP_GATHER = np.minimum(
    3125 * np.arange(_NW)[:, None, None]
    + 125 * np.arange(_NB)[None, :, None]
    + np.arange(_RBP)[None, None, :],
    _M - 1,
).reshape(_NW, _NB * _RBP).astype(np.int32)


def _sqrt16(x):
    x = jnp.maximum(x, 1e-30)
    i = lax.bitcast_convert_type(x, jnp.int32)
    y = lax.bitcast_convert_type(jnp.int32(0x5F3759DF) - (i >> 1), jnp.float32)
    for _ in range(2):
        y = y * (1.5 - 0.5 * x * y * y)
    return x * y


def _sc_body(idx_hbm, grad_hbm, emb_hbm, step_hbm, mem_hbm, pow_hbm,
             oemb_hbm, ostep_hbm, omem_hbm, opow_hbm,
             idx_v, pos_v, sub_v, tch_v, cidx_v, cnt_v, step_v,
             acc_v, emb_v, mem_v, pow_v, gbuf_v, dsem):
    cid = lax.axis_index("c")
    sid = lax.axis_index("s")
    wid = sid * 2 + cid
    lo = wid * _RPT

    iota = lax.iota(jnp.int32, 16)
    zeros16 = jnp.zeros((16,), jnp.float32)
    ones16 = jnp.ones((16,), jnp.float32)
    izeros16 = jnp.zeros((16,), jnp.int32)

    pltpu.sync_copy(idx_hbm, idx_v.at[pl.ds(0, _B)])
    pltpu.sync_copy(step_hbm.at[wid], step_v.at[pl.ds(0, _NB * _RBP)])

    @pl.loop(0, (_NB * _RBP) // 16, step=8)
    def _(k0):
        for u in range(8):
            cnt_v[pl.ds((k0 + u) * 16, 16)] = zeros16

    @pl.loop(0, sub_v.shape[0] // 16, step=8)
    def _(k0):
        for u in range(8):
            sub_v[pl.ds((k0 + u) * 16, 16)] = izeros16

    def scan_body(g4, base):
        for u in range(4):
            g = g4 * 4 + u
            v = idx_v[pl.ds(g * 16, 16)]
            vl = v - lo
            m = (vl >= 0) & (vl < _RPT)
            mi = m.astype(jnp.int32)
            blk = vl // _RB
            slot = blk * _RBP + (vl - blk * _RB)
            plsc.addupdate_scatter(cnt_v, [slot], ones16, mask=m)
            rank = plsc.cumsum(mi) - mi
            plsc.store_scatter(pos_v, [base + rank], iota + g * 16, mask=m)
            base = base + plsc.all_reduce_population_count(m)
        return base

    n_own_v = lax.fori_loop(0, _B // 64, scan_body, izeros16)
    n_own = n_own_v[0]
    nvec4 = (n_own + 63) // 64

    @pl.loop(0, _NB)
    def _blk(b):
        blk_lo = lo + b * _RB
        ci1 = pltpu.async_copy(emb_hbm.at[pl.ds(blk_lo, _RB)], emb_v, dsem)
        ci2 = pltpu.async_copy(mem_hbm.at[pl.ds(blk_lo, _RB)], mem_v, dsem)
        ci3 = pltpu.async_copy(pow_hbm.at[pl.ds(blk_lo, _RB)], pow_v, dsem)
        ci1.wait()
        ci2.wait()
        ci3.wait()

        def touch_body(k, base):
            off = b * _RBP + k * 16
            vc = cnt_v[pl.ds(off, 16)]
            m = vc > 0.0
            sv = step_v[pl.ds(off, 16)]
            step_v[pl.ds(off, 16)] = jnp.where(m, sv + 1.0, sv)
            mi = m.astype(jnp.int32)
            rank = plsc.cumsum(mi) - mi
            plsc.store_scatter(tch_v, [base + rank], iota + k * 16, mask=m)
            return base + plsc.all_reduce_population_count(m)

        n_t_v = lax.fori_loop(0, _RBP // 16, touch_body, izeros16)
        n_t = n_t_v[0]
        n_tg = (n_t + 15) // 16

        def zacc_body(t, _t):
            valid = (t * 16 + iota) < n_t_v
            lrv = tch_v[pl.ds(t * 16, 16)]
            for cc in range(_D):
                ccv = jnp.full((16,), cc, jnp.int32)
                plsc.store_scatter(acc_v, [lrv, ccv], zeros16, mask=valid)
            return _t

        lax.fori_loop(0, n_tg, zacc_body, 0)

        def filt(h4, base):
            for u in range(4):
                h = h4 * 4 + u
                pv = pos_v[pl.ds(h * 16, 16)]
                valid = (h * 16 + iota) < n_own_v
                rows = plsc.load_gather(idx_v, [pv], mask=valid)
                m2 = valid & (rows >= blk_lo) & (rows < blk_lo + _RB)
                mi = m2.astype(jnp.int32)
                rank = plsc.cumsum(mi) - mi
                plsc.store_scatter(sub_v, [base + rank], pv, mask=m2)
                base = base + plsc.all_reduce_population_count(m2)
            return base

        n_sub_v = lax.fori_loop(0, nvec4, filt, izeros16)
        n_sub = n_sub_v[0]

        def chunk_body(c, _c):
            nsc = jnp.minimum(n_sub - c * _CHUNK + 15, _CHUNK) // 16

            def sub_dma(k, _k):
                cidx_v[pl.ds(0, 16)] = sub_v[pl.ds(c * _CHUNK + k * 16, 16)]
                pltpu.sync_copy(grad_hbm.at[cidx_v.at[pl.ds(0, 16)]],
                                gbuf_v.at[pl.ds(k * 16, 16)])
                return _k

            lax.fori_loop(0, nsc, sub_dma, 0)

            @pl.loop(0, _CHUNK // 16)
            def _(e):
                ent = c * _CHUNK + e * 16
                valid = (ent + iota) < n_sub_v
                pv = sub_v[pl.ds(ent, 16)]
                rows = plsc.load_gather(idx_v, [pv], mask=valid)
                lr = rows - blk_lo
                gid = iota + e * 16
                for cc in range(_D):
                    ccv = jnp.full((16,), cc, jnp.int32)
                    vals = plsc.load_gather(gbuf_v, [gid, ccv])
                    plsc.addupdate_scatter(acc_v, [lr, ccv], vals, mask=valid)

            return _c

        nch = (n_sub + _CHUNK - 1) // _CHUNK
        lax.fori_loop(0, nch, chunk_body, 0)

        def rows_body(t, _t):
            valid = (t * 16 + iota) < n_t_v
            lrv = tch_v[pl.ds(t * 16, 16)]
            slots = lrv + b * _RBP
            cntv = plsc.load_gather(cnt_v, [slots], mask=valid)
            snew = plsc.load_gather(step_v, [slots], mask=valid)
            inv_c = ones16 / cntv
            c1 = 1.0 - jnp.exp(snew * _LN_B1)
            c2 = 1.0 - jnp.exp(snew * _LN_B2)

            @pl.loop(0, _D, step=16)
            def _(cc0):
                for u in range(16):
                    ccv = jnp.full((16,), cc0 + u, jnp.int32)
                    g = plsc.load_gather(acc_v, [lrv, ccv], mask=valid) * inv_c
                    mv = plsc.load_gather(mem_v, [lrv, ccv], mask=valid)
                    pv = plsc.load_gather(pow_v, [lrv, ccv], mask=valid)
                    um = BETA1 * mv + (1.0 - BETA1) * g
                    up = BETA2 * pv + (1.0 - BETA2) * (g * g)
                    std = LR * (um / c1) / (_sqrt16(up / c2) + EPS)
                    plsc.addupdate_scatter(emb_v, [lrv, ccv], -std, mask=valid)
                    plsc.store_scatter(mem_v, [lrv, ccv], um, mask=valid)
                    plsc.store_scatter(pow_v, [lrv, ccv], up, mask=valid)

            return _t

        lax.fori_loop(0, n_tg, rows_body, 0)

        co1 = pltpu.async_copy(emb_v, oemb_hbm.at[pl.ds(blk_lo, _RB)], dsem)
        co2 = pltpu.async_copy(mem_v, omem_hbm.at[pl.ds(blk_lo, _RB)], dsem)
        co3 = pltpu.async_copy(pow_v, opow_hbm.at[pl.ds(blk_lo, _RB)], dsem)
        co1.wait()
        co2.wait()
        co3.wait()

    pltpu.sync_copy(step_v.at[pl.ds(0, _NB * _RBP)], ostep_hbm.at[wid])


def kernel(idx, grad, emb, state_step, state_mem, state_power):
    step2d = state_step[_STEP_GATHER]
    mesh = plsc.VectorSubcoreMesh(core_axis_name="c", subcore_axis_name="s")
    out_type = [
        jax.ShapeDtypeStruct((_M, _D), jnp.float32),
        jax.ShapeDtypeStruct((_NW, _NB * _RBP), jnp.float32),
        jax.ShapeDtypeStruct((_M, _D), jnp.float32),
        jax.ShapeDtypeStruct((_M, _D), jnp.float32),
    ]
    scratch = [
        pltpu.VMEM((_B + 16,), jnp.int32),
        pltpu.VMEM((_B + 256,), jnp.int32),
        pltpu.VMEM((_B + _CHUNK + 16,), jnp.int32),
        pltpu.VMEM((_RBP + 16,), jnp.int32),
        pltpu.VMEM((_CHUNK,), jnp.int32),
        pltpu.VMEM((_NB * _RBP + 16,), jnp.float32),
        pltpu.VMEM((_NB * _RBP + 16,), jnp.float32),
        pltpu.VMEM((_RB, _D), jnp.float32),
        pltpu.VMEM((_RB, _D), jnp.float32),
        pltpu.VMEM((_RB, _D), jnp.float32),
        pltpu.VMEM((_RB, _D), jnp.float32),
        pltpu.VMEM((_CHUNK, _D), jnp.float32),
        pltpu.SemaphoreType.DMA,
    ]
    f = pl.kernel(_sc_body, out_type=out_type, mesh=mesh,
                  scratch_types=scratch,
                  compiler_params=pltpu.CompilerParams(
                      use_tc_tiling_on_sc=False,
                      needs_layout_passes=False))
    oemb, ostep2d, omem, opow = f(idx, grad, emb, step2d,
                                  state_mem, state_power)
    new_step = ostep2d.reshape(_NW, _NB, _RBP)[:, :, :_RB].reshape(_M)
    return oemb, new_step, omem, opow


# speedup vs baseline: 2.3805x; 1.0133x over previous
import numpy as np
import jax
import jax.numpy as jnp
from jax import lax
from jax.experimental import pallas as pl
from jax.experimental.pallas import tpu as pltpu
from jax.experimental.pallas import tpu_sc as plsc

BETA1 = 0.9
BETA2 = 0.999
EPS = 1e-08
LR = 0.001

_M = 100000
_D = 64
_B = 16384
_NW = 32
_RPT = _M // _NW
_NB = 25
_RB = _RPT // _NB
_RBP = 128
_CHUNK = 128
_LN_B1 = float(np.log(BETA1))
_LN_B2 = float(np.log(BETA2))

_STEP_GATHER = np.minimum(
    3125 * np.arange(_NW)[:, None, None]
    + 125 * np.arange(_NB)[None, :, None]
    + np.arange(_RBP)[None, None, :],
    _M - 1,
).reshape(_NW, _NB * _RBP).astype(np.int32)


def _sqrt16(x):
    x = jnp.maximum(x, 1e-30)
    i = lax.bitcast_convert_type(x, jnp.int32)
    y = lax.bitcast_convert_type(jnp.int32(0x5F3759DF) - (i >> 1), jnp.float32)
    for _ in range(2):
        y = y * (1.5 - 0.5 * x * y * y)
    return x * y


def _sc_body(idx_hbm, grad_hbm, emb_hbm, step_hbm, mem_hbm, pow_hbm,
             oemb_hbm, ostep_hbm, omem_hbm, opow_hbm,
             idx_v, pos_v, sub_v, tch_v, cidx_v, cnt_v, step_v,
             acc_v, emb_v, mem_v, pow_v, gbuf_v, dsem, gsem):
    cid = lax.axis_index("c")
    sid = lax.axis_index("s")
    wid = sid * 2 + cid
    lo = wid * _RPT

    iota = lax.iota(jnp.int32, 16)
    zeros16 = jnp.zeros((16,), jnp.float32)
    ones16 = jnp.ones((16,), jnp.float32)
    izeros16 = jnp.zeros((16,), jnp.int32)

    pltpu.sync_copy(idx_hbm, idx_v.at[pl.ds(0, _B)])
    pltpu.sync_copy(step_hbm.at[wid], step_v.at[pl.ds(0, _NB * _RBP)])

    @pl.loop(0, (_NB * _RBP) // 16, step=8)
    def _(k0):
        for u in range(8):
            cnt_v[pl.ds((k0 + u) * 16, 16)] = zeros16

    @pl.loop(0, sub_v.shape[0] // 16, step=8)
    def _(k0):
        for u in range(8):
            sub_v[pl.ds((k0 + u) * 16, 16)] = izeros16

    def scan_body(g4, base):
        for u in range(4):
            g = g4 * 4 + u
            v = idx_v[pl.ds(g * 16, 16)]
            vl = v - lo
            m = (vl >= 0) & (vl < _RPT)
            mi = m.astype(jnp.int32)
            blk = vl // _RB
            slot = blk * _RBP + (vl - blk * _RB)
            plsc.addupdate_scatter(cnt_v, [slot], ones16, mask=m)
            rank = plsc.cumsum(mi) - mi
            plsc.store_scatter(pos_v, [base + rank], iota + g * 16, mask=m)
            base = base + plsc.all_reduce_population_count(m)
        return base

    n_own_v = lax.fori_loop(0, _B // 64, scan_body, izeros16)
    n_own = n_own_v[0]
    nvec4 = (n_own + 63) // 64

    @pl.loop(0, _NB)
    def _blk(b):
        blk_lo = lo + b * _RB
        ci1 = pltpu.async_copy(emb_hbm.at[pl.ds(blk_lo, _RB)], emb_v, dsem)
        ci2 = pltpu.async_copy(mem_hbm.at[pl.ds(blk_lo, _RB)], mem_v, dsem)
        ci3 = pltpu.async_copy(pow_hbm.at[pl.ds(blk_lo, _RB)], pow_v, dsem)
        ci1.wait()
        ci2.wait()
        ci3.wait()

        def touch_body(k, base):
            off = b * _RBP + k * 16
            vc = cnt_v[pl.ds(off, 16)]
            m = vc > 0.0
            sv = step_v[pl.ds(off, 16)]
            step_v[pl.ds(off, 16)] = jnp.where(m, sv + 1.0, sv)
            mi = m.astype(jnp.int32)
            rank = plsc.cumsum(mi) - mi
            plsc.store_scatter(tch_v, [base + rank], iota + k * 16, mask=m)
            return base + plsc.all_reduce_population_count(m)

        n_t_v = lax.fori_loop(0, _RBP // 16, touch_body, izeros16)
        n_t = n_t_v[0]
        n_tg = (n_t + 15) // 16

        def zacc_body(t, _t):
            valid = (t * 16 + iota) < n_t_v
            lrv = tch_v[pl.ds(t * 16, 16)]
            for cc in range(_D):
                ccv = jnp.full((16,), cc, jnp.int32)
                plsc.store_scatter(acc_v, [lrv, ccv], zeros16, mask=valid)
            return _t

        lax.fori_loop(0, n_tg, zacc_body, 0)

        def filt(h4, base):
            for u in range(4):
                h = h4 * 4 + u
                pv = pos_v[pl.ds(h * 16, 16)]
                valid = (h * 16 + iota) < n_own_v
                rows = plsc.load_gather(idx_v, [pv], mask=valid)
                m2 = valid & (rows >= blk_lo) & (rows < blk_lo + _RB)
                mi = m2.astype(jnp.int32)
                rank = plsc.cumsum(mi) - mi
                plsc.store_scatter(sub_v, [base + rank], pv, mask=m2)
                base = base + plsc.all_reduce_population_count(m2)
            return base

        n_sub_v = lax.fori_loop(0, nvec4, filt, izeros16)
        n_sub = n_sub_v[0]

        def chunk_body(c, _c):
            nsc = jnp.minimum(n_sub - c * _CHUNK + 15, _CHUNK) // 16

            def fire(k, _k):
                cidx_v[pl.ds(k * 16, 16)] = sub_v[pl.ds(c * _CHUNK + k * 16, 16)]
                pltpu.async_copy(grad_hbm.at[cidx_v.at[pl.ds(k * 16, 16)]],
                                 gbuf_v.at[pl.ds(k * 16, 16)], gsem)
                return _k

            def drain(k, _k):
                pltpu.make_async_copy(
                    grad_hbm.at[cidx_v.at[pl.ds(k * 16, 16)]],
                    gbuf_v.at[pl.ds(k * 16, 16)], gsem).wait()
                return _k

            lax.fori_loop(0, nsc, fire, 0)
            lax.fori_loop(0, nsc, drain, 0)

            @pl.loop(0, _CHUNK // 16)
            def _(e):
                ent = c * _CHUNK + e * 16
                valid = (ent + iota) < n_sub_v
                pv = sub_v[pl.ds(ent, 16)]
                rows = plsc.load_gather(idx_v, [pv], mask=valid)
                lr = rows - blk_lo
                gid = iota + e * 16
                for cc in range(_D):
                    ccv = jnp.full((16,), cc, jnp.int32)
                    vals = plsc.load_gather(gbuf_v, [gid, ccv])
                    plsc.addupdate_scatter(acc_v, [lr, ccv], vals, mask=valid)

            return _c

        nch = (n_sub + _CHUNK - 1) // _CHUNK
        lax.fori_loop(0, nch, chunk_body, 0)

        def rows_body(t, _t):
            valid = (t * 16 + iota) < n_t_v
            lrv = tch_v[pl.ds(t * 16, 16)]
            slots = lrv + b * _RBP
            cntv = plsc.load_gather(cnt_v, [slots], mask=valid)
            snew = plsc.load_gather(step_v, [slots], mask=valid)
            inv_c = ones16 / cntv
            c1 = 1.0 - jnp.exp(snew * _LN_B1)
            c2 = 1.0 - jnp.exp(snew * _LN_B2)

            @pl.loop(0, _D, step=16)
            def _(cc0):
                for u in range(16):
                    ccv = jnp.full((16,), cc0 + u, jnp.int32)
                    g = plsc.load_gather(acc_v, [lrv, ccv], mask=valid) * inv_c
                    mv = plsc.load_gather(mem_v, [lrv, ccv], mask=valid)
                    pv = plsc.load_gather(pow_v, [lrv, ccv], mask=valid)
                    um = BETA1 * mv + (1.0 - BETA1) * g
                    up = BETA2 * pv + (1.0 - BETA2) * (g * g)
                    std = LR * (um / c1) / (_sqrt16(up / c2) + EPS)
                    plsc.addupdate_scatter(emb_v, [lrv, ccv], -std, mask=valid)
                    plsc.store_scatter(mem_v, [lrv, ccv], um, mask=valid)
                    plsc.store_scatter(pow_v, [lrv, ccv], up, mask=valid)

            return _t

        lax.fori_loop(0, n_tg, rows_body, 0)

        co1 = pltpu.async_copy(emb_v, oemb_hbm.at[pl.ds(blk_lo, _RB)], dsem)
        co2 = pltpu.async_copy(mem_v, omem_hbm.at[pl.ds(blk_lo, _RB)], dsem)
        co3 = pltpu.async_copy(pow_v, opow_hbm.at[pl.ds(blk_lo, _RB)], dsem)
        co1.wait()
        co2.wait()
        co3.wait()

    pltpu.sync_copy(step_v.at[pl.ds(0, _NB * _RBP)], ostep_hbm.at[wid])


def kernel(idx, grad, emb, state_step, state_mem, state_power):
    step2d = state_step[_STEP_GATHER]
    mesh = plsc.VectorSubcoreMesh(core_axis_name="c", subcore_axis_name="s")
    out_type = [
        jax.ShapeDtypeStruct((_M, _D), jnp.float32),
        jax.ShapeDtypeStruct((_NW, _NB * _RBP), jnp.float32),
        jax.ShapeDtypeStruct((_M, _D), jnp.float32),
        jax.ShapeDtypeStruct((_M, _D), jnp.float32),
    ]
    scratch = [
        pltpu.VMEM((_B + 16,), jnp.int32),
        pltpu.VMEM((_B + 256,), jnp.int32),
        pltpu.VMEM((_B + _CHUNK + 16,), jnp.int32),
        pltpu.VMEM((_RBP + 16,), jnp.int32),
        pltpu.VMEM((_CHUNK,), jnp.int32),
        pltpu.VMEM((_NB * _RBP + 16,), jnp.float32),
        pltpu.VMEM((_NB * _RBP + 16,), jnp.float32),
        pltpu.VMEM((_RB, _D), jnp.float32),
        pltpu.VMEM((_RB, _D), jnp.float32),
        pltpu.VMEM((_RB, _D), jnp.float32),
        pltpu.VMEM((_RB, _D), jnp.float32),
        pltpu.VMEM((_CHUNK, _D), jnp.float32),
        pltpu.SemaphoreType.DMA,
        pltpu.SemaphoreType.DMA,
    ]
    f = pl.kernel(_sc_body, out_type=out_type, mesh=mesh,
                  scratch_types=scratch,
                  compiler_params=pltpu.CompilerParams(
                      use_tc_tiling_on_sc=False,
                      needs_layout_passes=False))
    oemb, ostep2d, omem, opow = f(idx, grad, emb, step2d,
                                  state_mem, state_power)
    new_step = ostep2d.reshape(_NW, _NB, _RBP)[:, :, :_RB].reshape(_M)
    return oemb, new_step, omem, opow
